# baseline (speedup 1.0000x reference)
import functools

import jax
import jax.numpy as jnp
from jax import lax
from jax.experimental import pallas as pl
from jax.experimental.pallas import tpu as pltpu
from jax.experimental.pallas import tpu_sc as plsc

BATCH = 16384
K = 16
NC = 2
NS = 16
L = 16
NW = NC * NS
BPW = BATCH // NW
CHUNK = 128
NCHUNK = BPW // CHUNK

_mesh = plsc.VectorSubcoreMesh(core_axis_name="c", subcore_axis_name="s")


@functools.partial(
    pl.kernel,
    out_type=jax.ShapeDtypeStruct((BATCH,), jnp.float32),
    mesh=_mesh,
    compiler_params=pltpu.CompilerParams(
        needs_layout_passes=False, use_tc_tiling_on_sc=False
    ),
)
def _mf_predict(uidx_hbm, iidx_hbm, wt_hbm, ht_hbm, out_hbm):
    def body(uidx, iidx, ucols, vcols, outv, sem_u, sem_v):
        wid = lax.axis_index("s") * NC + lax.axis_index("c")
        base = wid * BPW
        iota = lax.iota(jnp.int32, L)

        pltpu.sync_copy(uidx_hbm.at[pl.ds(base, BPW)], uidx)
        pltpu.sync_copy(iidx_hbm.at[pl.ds(base, BPW)], iidx)

        copies = []
        for d in range(K):
            for c in range(NCHUNK):
                sl = pl.ds(c * CHUNK, CHUNK)
                copies.append(pltpu.async_copy(
                    wt_hbm.at[d].at[uidx.at[sl]], ucols.at[d].at[sl], sem_u))
                copies.append(pltpu.async_copy(
                    ht_hbm.at[d].at[iidx.at[sl]], vcols.at[d].at[sl], sem_v))
        for cp in copies:
            cp.wait()

        def dot_body(g, carry):
            col0 = g * L
            acc = jnp.zeros((L,), jnp.float32)
            for d in range(K):
                u = ucols[d, pl.ds(col0, L)]
                v = vcols[d, pl.ds(col0, L)]
                acc = acc + u * v
            sig = 1.0 / (1.0 + jnp.exp(-acc))
            plsc.store_scatter(outv, [col0 + iota], sig)
            return carry

        lax.fori_loop(0, BPW // L, dot_body, 0)

        pltpu.sync_copy(outv, out_hbm.at[pl.ds(base, BPW)])

    pl.run_scoped(
        body,
        pltpu.VMEM((BPW,), jnp.int32),
        pltpu.VMEM((BPW,), jnp.int32),
        pltpu.VMEM((K, BPW), jnp.float32),
        pltpu.VMEM((K, BPW), jnp.float32),
        pltpu.VMEM((BPW,), jnp.float32),
        pltpu.SemaphoreType.DMA,
        pltpu.SemaphoreType.DMA,
    )


def kernel(x, W, H):
    return _mf_predict(x[:, 0], x[:, 1], W.T, H.T)


# speedup vs baseline: 19.5301x; 19.5301x over previous
import functools

import jax
import jax.numpy as jnp
from jax import lax
from jax.experimental import pallas as pl
from jax.experimental.pallas import tpu as pltpu
from jax.experimental.pallas import tpu_sc as plsc

BATCH = 16384
K = 16
NC = 2
NS = 16
L = 16
NW = NC * NS
BPW = BATCH // NW
NGRP = BPW // L
SLAB = 128

_mesh = plsc.VectorSubcoreMesh(core_axis_name="c", subcore_axis_name="s")


@functools.partial(
    pl.kernel,
    out_type=jax.ShapeDtypeStruct((BATCH,), jnp.float32),
    mesh=_mesh,
    compiler_params=pltpu.CompilerParams(
        needs_layout_passes=False,
        disable_bounds_checks=True,
    ),
)
def _mf_predict(uidx_hbm, iidx_hbm, wt_hbm, ht_hbm, out_hbm):
    def body(uidx, iidx, uslab, vslab, outv, sem):
        wid = lax.axis_index("s") * NC + lax.axis_index("c")
        base = wid * BPW
        iota = lax.iota(jnp.int32, L)

        pltpu.sync_copy(uidx_hbm.at[pl.ds(base, BPW)], uidx)
        pltpu.sync_copy(iidx_hbm.at[pl.ds(base, BPW)], iidx)

        def group_body(g, carry):
            uvec = uidx[pl.ds(g * L, L)]
            ivec = iidx[pl.ds(g * L, L)]
            ustart = (uvec >> 7) << 7
            istart = (ivec >> 7) << 7
            ulane = uvec & 127
            ilane = ivec & 127
            copies = []
            for j in range(L):
                us = pl.multiple_of(ustart[j], SLAB)
                hs = pl.multiple_of(istart[j], SLAB)
                copies.append(pltpu.async_copy(
                    wt_hbm.at[:, pl.ds(us, SLAB)], uslab.at[j], sem))
                copies.append(pltpu.async_copy(
                    ht_hbm.at[:, pl.ds(hs, SLAB)], vslab.at[j], sem))
            for cp in copies:
                cp.wait()
            acc = jnp.zeros((L,), jnp.float32)
            for d in range(K):
                dsplat = jnp.full((L,), d, jnp.int32)
                u = plsc.load_gather(uslab, [iota, dsplat, ulane])
                v = plsc.load_gather(vslab, [iota, dsplat, ilane])
                acc = acc + u * v
            sig = 1.0 / (1.0 + jnp.exp(-acc))
            outv[pl.ds(g * L, L)] = sig
            return carry

        lax.fori_loop(0, NGRP, group_body, 0)

        pltpu.sync_copy(outv, out_hbm.at[pl.ds(base, BPW)])

    pl.run_scoped(
        body,
        pltpu.VMEM((BPW,), jnp.int32),
        pltpu.VMEM((BPW,), jnp.int32),
        pltpu.VMEM((L, K, SLAB), jnp.float32),
        pltpu.VMEM((L, K, SLAB), jnp.float32),
        pltpu.VMEM((BPW,), jnp.float32),
        pltpu.SemaphoreType.DMA,
    )


def kernel(x, W, H):
    return _mf_predict(x[:, 0], x[:, 1], W.T, H.T)
